# baseline (speedup 1.0000x reference)
import functools

import jax
import jax.numpy as jnp
from jax import lax
from jax.experimental import pallas as pl
from jax.experimental.pallas import tpu as pltpu
from jax.experimental.pallas import tpu_sc as plsc

_B, _S, _D = 4, 4096, 1024
_NC, _NS = 2, 16
_NW = _NC * _NS
_SW = _S // _NW
_R = 8
_C = _SW // _R
_NV = _R * _D // 16


@functools.partial(
    pl.kernel,
    out_type=jax.ShapeDtypeStruct((_B, _S, _D), jnp.float32),
    mesh=plsc.VectorSubcoreMesh(core_axis_name="c", subcore_axis_name="s"),
    scratch_types=[
        pltpu.VMEM((3, _B, _R, _D), jnp.float32),
        pltpu.VMEM((3, _R, _D), jnp.float32),
        pltpu.SemaphoreType.DMA,
        pltpu.SemaphoreType.DMA,
        pltpu.SemaphoreType.DMA,
        pltpu.SemaphoreType.DMA,
        pltpu.SemaphoreType.DMA,
        pltpu.SemaphoreType.DMA,
    ],
    compiler_params=pltpu.CompilerParams(use_tc_tiling_on_sc=True),
)
def _sc_add(x_hbm, t_hbm, o_hbm, x_buf, t_buf,
            sem_i0, sem_i1, sem_i2, sem_o0, sem_o1, sem_o2):
    sems_in = (sem_i0, sem_i1, sem_i2)
    sems_out = (sem_o0, sem_o1, sem_o2)
    wid = lax.axis_index("s") * _NC + lax.axis_index("c")
    base_s = wid * _SW

    copies_in = [None, None, None]
    copies_out = [None, None, None]

    def start_in(c):
        q = c % 3
        s0 = base_s + c * _R
        copies_in[q] = [
            pltpu.async_copy(t_hbm.at[pl.ds(s0, _R), :], t_buf.at[q], sems_in[q]),
            pltpu.async_copy(x_hbm.at[:, pl.ds(s0, _R), :], x_buf.at[q],
                             sems_in[q]),
        ]

    def start_out(c):
        q = c % 3
        s0 = base_s + c * _R
        copies_out[q] = [
            pltpu.async_copy(x_buf.at[q], o_hbm.at[:, pl.ds(s0, _R), :],
                             sems_out[q]),
        ]

    def compute(c):
        q = c % 3

        def row(r, carry):
            def body(j, carry2):
                o16 = j * 16
                t = t_buf[q, r, pl.ds(o16, 16)]
                for b in range(_B):
                    x_buf[q, b, r, pl.ds(o16, 16)] = (
                        x_buf[q, b, r, pl.ds(o16, 16)] + t)
                return carry2

            return lax.fori_loop(0, _D // 16, body, carry, unroll=8)

        lax.fori_loop(0, _R, row, 0)

    start_in(0)
    start_in(1)
    for c in range(_C):
        if c + 2 < _C:
            if c >= 1:
                for d in copies_out[(c - 1) % 3]:
                    d.wait()
            start_in(c + 2)
        for d in copies_in[c % 3]:
            d.wait()
        compute(c)
        start_out(c)
    for d in copies_out[(_C - 2) % 3]:
        d.wait()
    for d in copies_out[(_C - 1) % 3]:
        d.wait()


def kernel(x, table):
    return _sc_add(x, table)
